# baseline (speedup 1.0000x reference)
import functools

import jax
import jax.numpy as jnp
from jax import lax
from jax.experimental import pallas as pl
from jax.experimental.pallas import tpu as pltpu
from jax.experimental.pallas import tpu_sc as plsc

N = 100000
E = 1600000
CH = 128
GRP = 5
E_PAD = 1638400
C_TOT = E_PAD // CH
K0, K1 = 280, 520
T_ACC = 102400
STRIPE = T_ACC // 16
ZR = 128
BLK = 2000


def _make_sc_pass(d, feat_split):
    mesh = plsc.VectorSubcoreMesh(core_axis_name="c", subcore_axis_name="s")

    @functools.partial(
        pl.kernel,
        mesh=mesh,
        out_type=jax.ShapeDtypeStruct((2, T_ACC, d), jnp.float32),
        compiler_params=pltpu.CompilerParams(use_tc_tiling_on_sc=False),
        scratch_types=[
            pltpu.VMEM((4, GRP, CH), jnp.int32),
            pltpu.VMEM((4, GRP, CH), jnp.int32),
            pltpu.VMEM((2, GRP, CH, d), jnp.float32),
            pltpu.VMEM((ZR, d), jnp.float32),
            pltpu.VMEM_SHARED((T_ACC, d), jnp.float32),
            pltpu.SemaphoreType.DMA((2,)),
            pltpu.SemaphoreType.DMA((2,)),
            pltpu.SemaphoreType.DMA((4,)),
        ],
    )
    def k(table_hbm, src_hbm, dst_hbm, zrow_hbm, out_hbm,
          src_v, dst_v, rows_v, zb_v, acc, sem_g, sem_s, sem_i):
        c = lax.axis_index("c")
        s = lax.axis_index("s")

        pltpu.sync_copy(zrow_hbm, zb_v)
        r0 = s * STRIPE
        for i in range(STRIPE // ZR):
            pltpu.sync_copy(zb_v, acc.at[pl.ds(r0 + i * ZR, ZR)])
        plsc.subcore_barrier()

        if feat_split:
            base = s * (C_TOT // 16)
            ng = (C_TOT // 16) // GRP
            ng4 = ng // 4
        else:
            base = jnp.where(c == 0, s * K0, 16 * K0 + s * K1)
            ng = jnp.where(c == 0, K0 // GRP, K1 // GRP)
            ng4 = jnp.where(c == 0, K0 // GRP // 4, K1 // GRP // 4)

        def issue_idx(g, slot):
            row = base + g * GRP
            pltpu.async_copy(src_hbm.at[pl.ds(row, GRP)],
                             src_v.at[slot], sem_i.at[slot])
            pltpu.async_copy(dst_hbm.at[pl.ds(row, GRP)],
                             dst_v.at[slot], sem_i.at[slot])

        def wait_idx(slot):
            pltpu.make_async_copy(src_hbm.at[pl.ds(0, GRP)],
                                  src_v.at[slot], sem_i.at[slot]).wait()
            pltpu.make_async_copy(dst_hbm.at[pl.ds(0, GRP)],
                                  dst_v.at[slot], sem_i.at[slot]).wait()
            if feat_split:
                for j in range(GRP):
                    for t in range(CH // 16):
                        v = src_v[slot, j, pl.ds(t * 16, 16)]
                        src_v[slot, j, pl.ds(t * 16, 16)] = v * 2 + c

        def issue_gathers(islot, rslot):
            for j in range(GRP):
                pltpu.async_copy(table_hbm.at[src_v.at[islot, j]],
                                 rows_v.at[rslot, j], sem_g.at[rslot])

        def wait_gathers(islot, rslot):
            for j in range(GRP):
                pltpu.make_async_copy(table_hbm.at[src_v.at[islot, j]],
                                      rows_v.at[rslot, j], sem_g.at[rslot]).wait()

        def issue_scatters(islot, rslot):
            for j in range(GRP):
                pltpu.async_copy(rows_v.at[rslot, j],
                                 acc.at[dst_v.at[islot, j]], sem_s.at[rslot],
                                 add=True)

        def wait_scatters(islot, rslot):
            for j in range(GRP):
                pltpu.make_async_copy(rows_v.at[rslot, j],
                                      acc.at[dst_v.at[islot, j]],
                                      sem_s.at[rslot]).wait()

        issue_idx(0, 0)
        wait_idx(0)
        issue_gathers(0, 0)
        issue_idx(1, 1)

        def body(q, carry):
            for u in range(4):
                g = q * 4 + u
                i4 = u
                i4n = (u + 1) % 4
                b = u % 2
                nb = 1 - b
                wait_gathers(i4, b)
                if u == 0:
                    @pl.when(g > 0)
                    def _():
                        wait_scatters(3, 1)
                else:
                    wait_scatters(i4 - 1, nb)
                issue_scatters(i4, b)

                @pl.when(g < ng - 1)
                def _():
                    wait_idx(i4n)
                    issue_gathers(i4n, nb)

                @pl.when(g < ng - 2)
                def _():
                    issue_idx(g + 2, (u + 2) % 4)

            return carry

        lax.fori_loop(0, ng4, body, 0)
        wait_scatters(3, 1)
        plsc.subcore_barrier()
        pltpu.sync_copy(acc.at[pl.ds(r0, STRIPE)], out_hbm.at[c, pl.ds(r0, STRIPE)])

    return k


_sc_l1 = _make_sc_pass(8, False)
_sc_l2 = _make_sc_pass(16, True)
_sc_l3 = _make_sc_pass(16, False)


def _full(shape):
    return pl.BlockSpec(shape, lambda i: tuple(0 for _ in shape))


def _rows(shape):
    def imap(i):
        return tuple(i if s == BLK else 0 for s in shape)
    return pl.BlockSpec(shape, imap)


def _tc1_body(p_ref, h0p_ref, wl1_ref, bl1_ref, wr1_ref, wl2_ref,
              h1_ref, g2_ref, rcnt_ref):
    p = p_ref[0] + p_ref[1]
    cnt = p[:, 6:7]
    rc = 1.0 / jnp.maximum(cnt, 1.0)
    mean = p[:, 0:6] * rc
    h0 = h0p_ref[:, 0:6]
    z = (jnp.dot(mean, wl1_ref[...], preferred_element_type=jnp.float32)
         + bl1_ref[...]
         + jnp.dot(h0, wr1_ref[...], preferred_element_type=jnp.float32))
    h1 = jnp.maximum(z, 0.0)
    h1_ref[...] = h1
    g2_ref[...] = jnp.dot(h1, wl2_ref[...], preferred_element_type=jnp.float32)
    rcnt_ref[...] = rc


def _tc2_body(a_ref, h1_ref, rcnt_ref, bl2_ref, wr2_ref, wl3_ref,
              h2_ref, g3_ref):
    m = jnp.concatenate([a_ref[0], a_ref[1]], axis=1) * rcnt_ref[...]
    z = m + bl2_ref[...] + jnp.dot(h1_ref[...], wr2_ref[...],
                                   preferred_element_type=jnp.float32)
    h2 = jnp.maximum(z, 0.0)
    h2_ref[...] = h2
    g3_ref[...] = jnp.dot(h2, wl3_ref[...], preferred_element_type=jnp.float32)


def _tc3_body(q_ref, h2_ref, rcnt_ref, bl3_ref, wr3_ref, out_ref):
    m = (q_ref[0] + q_ref[1]) * rcnt_ref[...]
    z = m + bl3_ref[...] + jnp.dot(h2_ref[...], wr3_ref[...],
                                   preferred_element_type=jnp.float32)
    out_ref[...] = jnp.maximum(z, 0.0)


def kernel(x, edge_index, Wl1, bl1, Wr1, Wl2, bl2, Wr2, Wl3, bl3, Wr3):
    src = edge_index[0]
    dst = edge_index[1]
    npad = E_PAD - E
    srcp = jnp.concatenate([src, jnp.zeros((npad,), jnp.int32)])
    dstp = jnp.concatenate([dst, jnp.full((npad,), N, jnp.int32)])
    src_e = srcp.reshape(C_TOT, CH)
    dst_e = dstp.reshape(C_TOT, CH)

    h0p = jnp.concatenate(
        [x[:, 4:10], jnp.ones((N, 1), jnp.float32), jnp.zeros((N, 1), jnp.float32)],
        axis=1)
    z8 = jnp.zeros((ZR, 8), jnp.float32)
    z16 = jnp.zeros((ZR, 16), jnp.float32)

    agg1 = _sc_l1(h0p, src_e, dst_e, z8)

    grid = (N // BLK,)
    h1, g2, rcnt = pl.pallas_call(
        _tc1_body,
        grid=grid,
        in_specs=[
            pl.BlockSpec((2, BLK, 8), lambda i: (0, i, 0)),
            _rows((BLK, 8)),
            _full((6, 32)), _full((1, 32)), _full((6, 32)), _full((32, 32)),
        ],
        out_specs=[_rows((BLK, 32)), _rows((BLK, 32)), _rows((BLK, 1))],
        out_shape=[
            jax.ShapeDtypeStruct((N, 32), jnp.float32),
            jax.ShapeDtypeStruct((N, 32), jnp.float32),
            jax.ShapeDtypeStruct((N, 1), jnp.float32),
        ],
    )(agg1, h0p, Wl1, bl1.reshape(1, 32), Wr1, Wl2)

    agg2 = _sc_l2(g2.reshape(2 * N, 16), src_e, dst_e, z16)

    h2, g3 = pl.pallas_call(
        _tc2_body,
        grid=grid,
        in_specs=[
            pl.BlockSpec((2, BLK, 16), lambda i: (0, i, 0)),
            _rows((BLK, 32)), _rows((BLK, 1)),
            _full((1, 32)), _full((32, 32)), _full((32, 16)),
        ],
        out_specs=[_rows((BLK, 32)), _rows((BLK, 16))],
        out_shape=[
            jax.ShapeDtypeStruct((N, 32), jnp.float32),
            jax.ShapeDtypeStruct((N, 16), jnp.float32),
        ],
    )(agg2, h1, rcnt, bl2.reshape(1, 32), Wr2, Wl3)

    agg3 = _sc_l3(g3, src_e, dst_e, z16)

    out = pl.pallas_call(
        _tc3_body,
        grid=grid,
        in_specs=[
            pl.BlockSpec((2, BLK, 16), lambda i: (0, i, 0)),
            _rows((BLK, 32)), _rows((BLK, 1)),
            _full((1, 16)), _full((32, 16)),
        ],
        out_specs=_rows((BLK, 16)),
        out_shape=jax.ShapeDtypeStruct((N, 16), jnp.float32),
    )(agg3, h2, rcnt, bl3.reshape(1, 16), Wr3)

    return out


# speedup vs baseline: 1.0792x; 1.0792x over previous
import functools

import jax
import jax.numpy as jnp
from jax import lax
from jax.experimental import pallas as pl
from jax.experimental.pallas import tpu as pltpu
from jax.experimental.pallas import tpu_sc as plsc

N = 100000
E = 1600000
CH = 128
GRP = 5
E_PAD = 1638400
C_TOT = E_PAD // CH
T_ACC = 102400
STRIPE = T_ACC // 16
ZR = 128
BLK = 2000


def _make_sc_pass(d, feat_split, k0=0, k1=0):
    mesh = plsc.VectorSubcoreMesh(core_axis_name="c", subcore_axis_name="s")

    @functools.partial(
        pl.kernel,
        mesh=mesh,
        out_type=jax.ShapeDtypeStruct((2, T_ACC, d), jnp.float32),
        compiler_params=pltpu.CompilerParams(use_tc_tiling_on_sc=False),
        scratch_types=[
            pltpu.VMEM((4, GRP * CH), jnp.int32),
            pltpu.VMEM((4, GRP, CH), jnp.int32),
            pltpu.VMEM((2, GRP * CH, d), jnp.float32),
            pltpu.VMEM((ZR, d), jnp.float32),
            pltpu.VMEM_SHARED((T_ACC, d), jnp.float32),
            pltpu.SemaphoreType.DMA((2,)),
            pltpu.SemaphoreType.DMA((2,)),
            pltpu.SemaphoreType.DMA((4,)),
        ],
    )
    def k(table_hbm, srcf_hbm, dst_hbm, zrow_hbm, out_hbm,
          src_v, dst_v, rows_v, zb_v, acc, sem_g, sem_s, sem_i):
        c = lax.axis_index("c")
        s = lax.axis_index("s")

        pltpu.sync_copy(zrow_hbm, zb_v)
        r0 = s * STRIPE
        for i in range(STRIPE // ZR):
            pltpu.sync_copy(zb_v, acc.at[pl.ds(r0 + i * ZR, ZR)])
        plsc.subcore_barrier()

        if feat_split:
            base = s * (C_TOT // 16)
            ng = (C_TOT // 16) // GRP
            ng4 = ng // 4
        else:
            base = jnp.where(c == 0, s * k0, 16 * k0 + s * k1)
            ng = jnp.where(c == 0, k0 // GRP, k1 // GRP)
            ng4 = jnp.where(c == 0, k0 // GRP // 4, k1 // GRP // 4)

        def issue_idx(g, slot):
            row = base + g * GRP
            pltpu.async_copy(srcf_hbm.at[pl.ds(row * CH, GRP * CH)],
                             src_v.at[slot], sem_i.at[slot])
            pltpu.async_copy(dst_hbm.at[pl.ds(row, GRP)],
                             dst_v.at[slot], sem_i.at[slot])

        def wait_idx(slot):
            pltpu.make_async_copy(srcf_hbm.at[pl.ds(0, GRP * CH)],
                                  src_v.at[slot], sem_i.at[slot]).wait()
            pltpu.make_async_copy(dst_hbm.at[pl.ds(0, GRP)],
                                  dst_v.at[slot], sem_i.at[slot]).wait()
            if feat_split:
                for t in range(GRP * CH // 16):
                    v = src_v[slot, pl.ds(t * 16, 16)]
                    src_v[slot, pl.ds(t * 16, 16)] = v * 2 + c

        def issue_gathers(islot, rslot):
            pltpu.async_copy(table_hbm.at[src_v.at[islot]],
                             rows_v.at[rslot], sem_g.at[rslot])

        def wait_gathers(islot, rslot):
            pltpu.make_async_copy(table_hbm.at[src_v.at[islot]],
                                  rows_v.at[rslot], sem_g.at[rslot]).wait()

        def issue_scatters(islot, rslot):
            for j in range(GRP):
                pltpu.async_copy(rows_v.at[rslot, pl.ds(j * CH, CH)],
                                 acc.at[dst_v.at[islot, j]], sem_s.at[rslot],
                                 add=True)

        def wait_scatters(islot, rslot):
            for j in range(GRP):
                pltpu.make_async_copy(rows_v.at[rslot, pl.ds(j * CH, CH)],
                                      acc.at[dst_v.at[islot, j]],
                                      sem_s.at[rslot]).wait()

        issue_idx(0, 0)
        wait_idx(0)
        issue_gathers(0, 0)
        issue_idx(1, 1)

        def body(q, carry):
            for u in range(4):
                g = q * 4 + u
                i4 = u
                i4n = (u + 1) % 4
                b = u % 2
                nb = 1 - b
                wait_gathers(i4, b)
                if u == 0:
                    @pl.when(g > 0)
                    def _():
                        wait_scatters(3, 1)
                else:
                    wait_scatters(i4 - 1, nb)
                issue_scatters(i4, b)

                @pl.when(g < ng - 1)
                def _():
                    wait_idx(i4n)
                    issue_gathers(i4n, nb)

                @pl.when(g < ng - 2)
                def _():
                    issue_idx(g + 2, (u + 2) % 4)

            return carry

        lax.fori_loop(0, ng4, body, 0)
        wait_scatters(3, 1)
        plsc.subcore_barrier()
        pltpu.sync_copy(acc.at[pl.ds(r0, STRIPE)], out_hbm.at[c, pl.ds(r0, STRIPE)])

    return k


_sc_l1 = _make_sc_pass(8, False, 580, 220)
_sc_l2 = _make_sc_pass(16, True)
_sc_l3 = _make_sc_pass(16, False, 620, 180)


def _full(shape):
    return pl.BlockSpec(shape, lambda i: tuple(0 for _ in shape))


def _rows(shape):
    def imap(i):
        return tuple(i if s == BLK else 0 for s in shape)
    return pl.BlockSpec(shape, imap)


def _tc1_body(p_ref, h0p_ref, wl1_ref, bl1_ref, wr1_ref, wl2_ref,
              h1_ref, g2_ref, rcnt_ref):
    p = p_ref[0] + p_ref[1]
    cnt = p[:, 6:7]
    rc = 1.0 / jnp.maximum(cnt, 1.0)
    mean = p[:, 0:6] * rc
    h0 = h0p_ref[:, 0:6]
    z = (jnp.dot(mean, wl1_ref[...], preferred_element_type=jnp.float32)
         + bl1_ref[...]
         + jnp.dot(h0, wr1_ref[...], preferred_element_type=jnp.float32))
    h1 = jnp.maximum(z, 0.0)
    h1_ref[...] = h1
    g2_ref[...] = jnp.dot(h1, wl2_ref[...], preferred_element_type=jnp.float32)
    rcnt_ref[...] = rc


def _tc2_body(a_ref, h1_ref, rcnt_ref, bl2_ref, wr2_ref, wl3_ref,
              h2_ref, g3_ref):
    m = jnp.concatenate([a_ref[0], a_ref[1]], axis=1) * rcnt_ref[...]
    z = m + bl2_ref[...] + jnp.dot(h1_ref[...], wr2_ref[...],
                                   preferred_element_type=jnp.float32)
    h2 = jnp.maximum(z, 0.0)
    h2_ref[...] = h2
    g3_ref[...] = jnp.dot(h2, wl3_ref[...], preferred_element_type=jnp.float32)


def _tc3_body(q_ref, h2_ref, rcnt_ref, bl3_ref, wr3_ref, out_ref):
    m = (q_ref[0] + q_ref[1]) * rcnt_ref[...]
    z = m + bl3_ref[...] + jnp.dot(h2_ref[...], wr3_ref[...],
                                   preferred_element_type=jnp.float32)
    out_ref[...] = jnp.maximum(z, 0.0)


def kernel(x, edge_index, Wl1, bl1, Wr1, Wl2, bl2, Wr2, Wl3, bl3, Wr3):
    src = edge_index[0]
    dst = edge_index[1]
    npad = E_PAD - E
    srcp = jnp.concatenate([src, jnp.zeros((npad,), jnp.int32)])
    dstp = jnp.concatenate([dst, jnp.full((npad,), N, jnp.int32)])
    dst_e = dstp.reshape(C_TOT, CH)

    h0p = jnp.concatenate(
        [x[:, 4:10], jnp.ones((N, 1), jnp.float32), jnp.zeros((N, 1), jnp.float32)],
        axis=1)
    z8 = jnp.zeros((ZR, 8), jnp.float32)
    z16 = jnp.zeros((ZR, 16), jnp.float32)

    agg1 = _sc_l1(h0p, srcp, dst_e, z8)

    grid = (N // BLK,)
    h1, g2, rcnt = pl.pallas_call(
        _tc1_body,
        grid=grid,
        in_specs=[
            pl.BlockSpec((2, BLK, 8), lambda i: (0, i, 0)),
            _rows((BLK, 8)),
            _full((6, 32)), _full((1, 32)), _full((6, 32)), _full((32, 32)),
        ],
        out_specs=[_rows((BLK, 32)), _rows((BLK, 32)), _rows((BLK, 1))],
        out_shape=[
            jax.ShapeDtypeStruct((N, 32), jnp.float32),
            jax.ShapeDtypeStruct((N, 32), jnp.float32),
            jax.ShapeDtypeStruct((N, 1), jnp.float32),
        ],
    )(agg1, h0p, Wl1, bl1.reshape(1, 32), Wr1, Wl2)

    agg2 = _sc_l2(g2.reshape(2 * N, 16), srcp, dst_e, z16)

    h2, g3 = pl.pallas_call(
        _tc2_body,
        grid=grid,
        in_specs=[
            pl.BlockSpec((2, BLK, 16), lambda i: (0, i, 0)),
            _rows((BLK, 32)), _rows((BLK, 1)),
            _full((1, 32)), _full((32, 32)), _full((32, 16)),
        ],
        out_specs=[_rows((BLK, 32)), _rows((BLK, 16))],
        out_shape=[
            jax.ShapeDtypeStruct((N, 32), jnp.float32),
            jax.ShapeDtypeStruct((N, 16), jnp.float32),
        ],
    )(agg2, h1, rcnt, bl2.reshape(1, 32), Wr2, Wl3)

    agg3 = _sc_l3(g3, srcp, dst_e, z16)

    out = pl.pallas_call(
        _tc3_body,
        grid=grid,
        in_specs=[
            pl.BlockSpec((2, BLK, 16), lambda i: (0, i, 0)),
            _rows((BLK, 32)), _rows((BLK, 1)),
            _full((1, 16)), _full((32, 16)),
        ],
        out_specs=_rows((BLK, 16)),
        out_shape=jax.ShapeDtypeStruct((N, 16), jnp.float32),
    )(agg3, h2, rcnt, bl3.reshape(1, 16), Wr3)

    return out
